# Initial kernel scaffold; baseline (speedup 1.0000x reference)
import jax
import jax.numpy as jnp
from jax.experimental import pallas as pl


def kernel(x, conv3_w, conv3_b, reg_w, reg_b, cls_w, cls_b):
    raise NotImplementedError("write your pallas kernel here")


# speedup vs baseline: 1.0994x; 1.0994x over previous
import jax
import jax.numpy as jnp
from jax.experimental import pallas as pl

_B, _C, _H, _W = 4, 512, 64, 64
_HW = _H * _W
_M = 512
_R = _HW // _M
_P = 128


def _rpn_body(xs_ref, w9_ref, b3_ref, wh_ref, bh_ref, out_ref):
    r = pl.program_id(1)
    start = r * _M + _P
    acc = jnp.zeros((_M, _C), jnp.float32)
    for k in range(9):
        dy, dx = k // 3 - 1, k % 3 - 1
        src = xs_ref[dx + 1, 0, pl.ds(start + dy * _W, _M), :]
        acc += jnp.dot(src, w9_ref[k], preferred_element_type=jnp.float32)
    h = jnp.maximum(acc + b3_ref[0][None, :], 0.0).astype(jnp.bfloat16)
    o = jnp.dot(h, wh_ref[...], preferred_element_type=jnp.float32)
    out_ref[0] = o + bh_ref[0][None, :]


@jax.jit
def kernel(x, conv3_w, conv3_b, reg_w, reg_b, cls_w, cls_b):
    B = x.shape[0]
    xt = jnp.transpose(x, (0, 2, 3, 1)).reshape(B, _HW, _C)
    wcol = (jnp.arange(_HW) % _W)[None, :, None]
    shifted = []
    for dx in (-1, 0, 1):
        xm = xt
        if dx == -1:
            xm = jnp.where(wcol == _W - 1, 0.0, xm)
        elif dx == 1:
            xm = jnp.where(wcol == 0, 0.0, xm)
        xm = jnp.roll(xm, -dx, axis=1) if dx else xm
        shifted.append(jnp.pad(xm, ((0, 0), (_P, _P), (0, 0))))
    xs = jnp.stack(shifted).astype(jnp.bfloat16)
    w9 = jnp.transpose(conv3_w, (2, 3, 1, 0)).reshape(9, _C, _C)
    w9 = w9.astype(jnp.bfloat16)
    wh = jnp.concatenate(
        [reg_w.reshape(36, _C).T, cls_w.reshape(9, _C).T,
         jnp.zeros((_C, 64 - 45), jnp.float32)], axis=1).astype(jnp.bfloat16)
    bh = jnp.concatenate([reg_b, cls_b, jnp.zeros((64 - 45,), jnp.float32)])

    out = pl.pallas_call(
        _rpn_body,
        grid=(B, _R),
        in_specs=[
            pl.BlockSpec((3, 1, _HW + 2 * _P, _C), lambda b, r: (0, b, 0, 0)),
            pl.BlockSpec((9, _C, _C), lambda b, r: (0, 0, 0)),
            pl.BlockSpec((1, _C), lambda b, r: (0, 0)),
            pl.BlockSpec((_C, 64), lambda b, r: (0, 0)),
            pl.BlockSpec((1, 64), lambda b, r: (0, 0)),
        ],
        out_specs=pl.BlockSpec((1, _M, 64), lambda b, r: (b, r, 0)),
        out_shape=jax.ShapeDtypeStruct((B, _HW, 64), jnp.float32),
    )(xs, w9, conv3_b.reshape(1, _C), wh, bh.reshape(1, 64))

    box = out[:, :, :36].reshape(B, _HW * 9, 4)
    cls = out[:, :, 36:45].reshape(B, _HW * 9, 1)
    return (box, cls)


# speedup vs baseline: 1.1423x; 1.0390x over previous
import jax
import jax.numpy as jnp
from jax.experimental import pallas as pl

_B, _C, _H, _W = 4, 512, 64, 64
_HW = _H * _W
_M = 2048
_R = _HW // _M
_P = 128


def _rpn_body(xs_ref, w9_ref, b3_ref, wh_ref, bh_ref, out_ref):
    r = pl.program_id(1)
    start = r * _M + _P
    acc = jnp.zeros((_M, _C), jnp.float32)
    for k in range(9):
        dy, dx = k // 3 - 1, k % 3 - 1
        src = xs_ref[dx + 1, 0, pl.ds(start + dy * _W, _M), :]
        acc += jnp.dot(src, w9_ref[k], preferred_element_type=jnp.float32)
    h = jnp.maximum(acc + b3_ref[0][None, :], 0.0).astype(jnp.bfloat16)
    o = jnp.dot(h, wh_ref[...], preferred_element_type=jnp.float32)
    out_ref[0] = o + bh_ref[0][None, :]


@jax.jit
def kernel(x, conv3_w, conv3_b, reg_w, reg_b, cls_w, cls_b):
    B = x.shape[0]
    xt = jnp.transpose(x, (0, 2, 3, 1)).reshape(B, _HW, _C)
    wcol = (jnp.arange(_HW) % _W)[None, :, None]
    shifted = []
    for dx in (-1, 0, 1):
        xm = xt
        if dx == -1:
            xm = jnp.where(wcol == _W - 1, 0.0, xm)
        elif dx == 1:
            xm = jnp.where(wcol == 0, 0.0, xm)
        xm = jnp.roll(xm, -dx, axis=1) if dx else xm
        shifted.append(jnp.pad(xm, ((0, 0), (_P, _P), (0, 0))))
    xs = jnp.stack(shifted).astype(jnp.bfloat16)
    w9 = jnp.transpose(conv3_w, (2, 3, 1, 0)).reshape(9, _C, _C)
    w9 = w9.astype(jnp.bfloat16)
    wh = jnp.concatenate(
        [reg_w.reshape(36, _C).T, cls_w.reshape(9, _C).T,
         jnp.zeros((_C, 64 - 45), jnp.float32)], axis=1).astype(jnp.bfloat16)
    bh = jnp.concatenate([reg_b, cls_b, jnp.zeros((64 - 45,), jnp.float32)])

    out = pl.pallas_call(
        _rpn_body,
        grid=(B, _R),
        in_specs=[
            pl.BlockSpec((3, 1, _HW + 2 * _P, _C), lambda b, r: (0, b, 0, 0)),
            pl.BlockSpec((9, _C, _C), lambda b, r: (0, 0, 0)),
            pl.BlockSpec((1, _C), lambda b, r: (0, 0)),
            pl.BlockSpec((_C, 64), lambda b, r: (0, 0)),
            pl.BlockSpec((1, 64), lambda b, r: (0, 0)),
        ],
        out_specs=pl.BlockSpec((1, _M, 64), lambda b, r: (b, r, 0)),
        out_shape=jax.ShapeDtypeStruct((B, _HW, 64), jnp.float32),
    )(xs, w9, conv3_b.reshape(1, _C), wh, bh.reshape(1, 64))

    box = out[:, :, :36].reshape(B, _HW * 9, 4)
    cls = out[:, :, 36:45].reshape(B, _HW * 9, 1)
    return (box, cls)


# speedup vs baseline: 1.1461x; 1.0033x over previous
import jax
import jax.numpy as jnp
from jax.experimental import pallas as pl

_B, _C, _H, _W = 4, 512, 64, 64
_HW = _H * _W
_M = 4096
_R = _HW // _M
_P = 128


def _rpn_body(xs_ref, w9_ref, b3_ref, wh_ref, bh_ref, out_ref):
    r = pl.program_id(1)
    start = r * _M + _P
    acc = jnp.zeros((_M, _C), jnp.float32)
    for k in range(9):
        dy, dx = k // 3 - 1, k % 3 - 1
        src = xs_ref[dx + 1, 0, pl.ds(start + dy * _W, _M), :]
        acc += jnp.dot(src, w9_ref[k], preferred_element_type=jnp.float32)
    h = jnp.maximum(acc + b3_ref[0][None, :], 0.0).astype(jnp.bfloat16)
    o = jnp.dot(h, wh_ref[...], preferred_element_type=jnp.float32)
    out_ref[0] = o + bh_ref[0][None, :]


@jax.jit
def kernel(x, conv3_w, conv3_b, reg_w, reg_b, cls_w, cls_b):
    B = x.shape[0]
    xt = jnp.transpose(x, (0, 2, 3, 1)).reshape(B, _HW, _C)
    wcol = (jnp.arange(_HW) % _W)[None, :, None]
    shifted = []
    for dx in (-1, 0, 1):
        xm = xt
        if dx == -1:
            xm = jnp.where(wcol == _W - 1, 0.0, xm)
        elif dx == 1:
            xm = jnp.where(wcol == 0, 0.0, xm)
        xm = jnp.roll(xm, -dx, axis=1) if dx else xm
        shifted.append(jnp.pad(xm, ((0, 0), (_P, _P), (0, 0))))
    xs = jnp.stack(shifted).astype(jnp.bfloat16)
    w9 = jnp.transpose(conv3_w, (2, 3, 1, 0)).reshape(9, _C, _C)
    w9 = w9.astype(jnp.bfloat16)
    wh = jnp.concatenate(
        [reg_w.reshape(36, _C).T, cls_w.reshape(9, _C).T,
         jnp.zeros((_C, 64 - 45), jnp.float32)], axis=1).astype(jnp.bfloat16)
    bh = jnp.concatenate([reg_b, cls_b, jnp.zeros((64 - 45,), jnp.float32)])

    out = pl.pallas_call(
        _rpn_body,
        grid=(B, _R),
        in_specs=[
            pl.BlockSpec((3, 1, _HW + 2 * _P, _C), lambda b, r: (0, b, 0, 0)),
            pl.BlockSpec((9, _C, _C), lambda b, r: (0, 0, 0)),
            pl.BlockSpec((1, _C), lambda b, r: (0, 0)),
            pl.BlockSpec((_C, 64), lambda b, r: (0, 0)),
            pl.BlockSpec((1, 64), lambda b, r: (0, 0)),
        ],
        out_specs=pl.BlockSpec((1, _M, 64), lambda b, r: (b, r, 0)),
        out_shape=jax.ShapeDtypeStruct((B, _HW, 64), jnp.float32),
    )(xs, w9, conv3_b.reshape(1, _C), wh, bh.reshape(1, 64))

    box = out[:, :, :36].reshape(B, _HW * 9, 4)
    cls = out[:, :, 36:45].reshape(B, _HW * 9, 1)
    return (box, cls)
